# Initial kernel scaffold; baseline (speedup 1.0000x reference)
import jax
import jax.numpy as jnp
from jax.experimental import pallas as pl


def kernel(x, embeddings):
    raise NotImplementedError("write your pallas kernel here")


# speedup vs baseline: 1.0315x; 1.0315x over previous
import functools

import jax
import jax.numpy as jnp
from jax import lax
from jax.experimental import pallas as pl
from jax.experimental.pallas import tpu as pltpu
from jax.experimental.pallas import tpu_sc as plsc

K = 8192
D = 8
N = 32768
BN = 1024
NT = N // BN

DP = 128
NW = 32
RPW = N // NW
IDX_CHUNK = 128


def _dist_argmin_body(x_ref, embt_ref, idx_ref):
    x = x_ref[...]
    embt = embt_ref[...]
    xsq = jnp.sum(x * x, axis=1, keepdims=True)
    esq = jnp.sum(embt * embt, axis=0, keepdims=True)
    cross = lax.dot_general(x, embt, (((1,), (0,)), ((), ())),
                            preferred_element_type=jnp.float32)
    d2 = (xsq + esq) - 2.0 * cross
    dist = jnp.sqrt(jnp.maximum(d2, 0.0))
    H = K // 2
    d0 = dist[:, :H]
    d1 = dist[:, H:]
    lanes = lax.broadcasted_iota(jnp.int32, (BN, H), 1)
    m0 = jnp.min(d0, axis=1, keepdims=True)
    i0 = jnp.min(jnp.where(d0 == m0, lanes, K), axis=1)
    m1 = jnp.min(d1, axis=1, keepdims=True)
    i1 = jnp.min(jnp.where(d1 == m1, lanes + H, K), axis=1)
    m0r = m0.astype(jnp.bfloat16).astype(jnp.float32)
    idx = jnp.where(m1[:, 0] < m0r[:, 0], i1, i0)
    idx_ref[0, 0, :] = idx


def _tc_dist_argmin(xf, embt):
    return pl.pallas_call(
        _dist_argmin_body,
        grid=(NT,),
        in_specs=[
            pl.BlockSpec((BN, DP), lambda i: (i, 0)),
            pl.BlockSpec((DP, K), lambda i: (0, 0)),
        ],
        out_specs=pl.BlockSpec((1, 1, BN), lambda i: (i, 0, 0)),
        out_shape=jax.ShapeDtypeStruct((NT, 1, BN), jnp.int32),
        compiler_params=pltpu.CompilerParams(
            dimension_semantics=("arbitrary",),
        ),
    )(xf, embt)


@functools.cache
def _make_sc_gather_loss():
    return pl.kernel(
        _sc_gather_loss_body,
        mesh=plsc.VectorSubcoreMesh(core_axis_name="c", subcore_axis_name="s"),
        out_type=[
            jax.ShapeDtypeStruct((N * D,), jnp.float32),
            jax.ShapeDtypeStruct((NW * 16,), jnp.float32),
        ],
        scratch_types=[
            pltpu.VMEM((RPW,), jnp.int32),
            pltpu.VMEM((RPW * D,), jnp.int32),
            pltpu.VMEM((RPW * D,), jnp.float32),
            pltpu.VMEM((RPW * D,), jnp.float32),
            pltpu.VMEM((16,), jnp.float32),
            pltpu.SemaphoreType.DMA,
        ],
    )


def _sc_gather_loss_body(emb_hbm, idx_hbm, xf_hbm, out_hbm, part_hbm,
                         idx_v, widx_v, rows_v, xf_v, acc_v, sem):
    wid = lax.axis_index("s") * 2 + lax.axis_index("c")
    base = wid * RPW
    pltpu.sync_copy(idx_hbm.at[pl.ds(base, RPW)], idx_v)
    pltpu.sync_copy(xf_hbm.at[pl.ds(base * D, RPW * D)], xf_v)

    lane = lax.iota(jnp.int32, 16)
    rep = lane >> 3
    off = lane & 7
    dnums = lax.GatherDimensionNumbers(
        offset_dims=(), collapsed_slice_dims=(0,), start_index_map=(0,))

    def tile_body(t, _):
        tile = idx_v[pl.ds(t * 16, 16)]
        for c in range(8):
            vals = lax.gather(
                tile, (2 * c + rep)[:, None], dnums, (1,),
                mode=lax.GatherScatterMode.PROMISE_IN_BOUNDS)
            widx_v[pl.ds(t * 128 + c * 16, 16)] = vals * 8 + off
        return 0

    lax.fori_loop(0, RPW // 16, tile_body, 0)

    copies = []
    for g in range(RPW * D // IDX_CHUNK):
        copies.append(pltpu.async_copy(
            emb_hbm.at[widx_v.at[pl.ds(g * IDX_CHUNK, IDX_CHUNK)]],
            rows_v.at[pl.ds(g * IDX_CHUNK, IDX_CHUNK)],
            sem,
        ))
    for c in copies:
        c.wait()

    def body(i, acc):
        dlt = rows_v[pl.ds(i * 16, 16)] - xf_v[pl.ds(i * 16, 16)]
        return acc + dlt * dlt

    acc = lax.fori_loop(0, RPW * D // 16, body, jnp.zeros((16,), jnp.float32))
    acc_v[...] = acc
    pltpu.sync_copy(rows_v, out_hbm.at[pl.ds(base * D, RPW * D)])
    pltpu.sync_copy(acc_v, part_hbm.at[pl.ds(wid * 16, 16)])


def kernel(x, embeddings):
    xf = x.reshape(N, D)
    xf_p = jnp.pad(xf, ((0, 0), (0, DP - D)))
    embt_p = jnp.pad(embeddings.T, ((0, DP - D), (0, 0)))
    idx3 = _tc_dist_argmin(xf_p, embt_p)
    indices = idx3.reshape(N)
    q, parts = _make_sc_gather_loss()(embeddings.reshape(-1), indices,
                                      xf.reshape(-1))
    m = jnp.sum(parts) / jnp.float32(N * D)
    loss = m + jnp.float32(0.25) * m
    quantized = q.reshape(x.shape)
    return (quantized, loss, indices)


# speedup vs baseline: 1.0636x; 1.0311x over previous
import functools

import jax
import jax.numpy as jnp
from jax import lax
from jax.experimental import pallas as pl
from jax.experimental.pallas import tpu as pltpu
from jax.experimental.pallas import tpu_sc as plsc

K = 8192
D = 8
N = 32768
BN = 1024
NT = N // BN

DP = 128
NW = 32
RPW = N // NW
IDX_CHUNK = 128


def _dist_argmin_body(x2_ref, embt_ref, idx_ref):
    x2 = x2_ref[...]
    embt = embt_ref[...]
    xsq = jnp.sum(x2 * x2, axis=1, keepdims=True) * 0.25
    esq = jnp.sum(embt * embt, axis=0, keepdims=True)
    cross2 = lax.dot_general(x2, embt, (((1,), (0,)), ((), ())),
                             preferred_element_type=jnp.float32)
    d2 = (xsq + esq) - cross2
    dist = jnp.sqrt(jnp.maximum(d2, 0.0))
    H = K // 2
    d0 = dist[:, :H]
    d1 = dist[:, H:]
    lanes = lax.broadcasted_iota(jnp.int32, (BN, H), 1)
    m0 = jnp.min(d0, axis=1, keepdims=True)
    i0 = jnp.min(jnp.where(d0 == m0, lanes, K), axis=1)
    m1 = jnp.min(d1, axis=1, keepdims=True)
    i1 = jnp.min(jnp.where(d1 == m1, lanes + H, K), axis=1)
    m0r = m0.astype(jnp.bfloat16).astype(jnp.float32)
    idx = jnp.where(m1[:, 0] < m0r[:, 0], i1, i0)
    idx_ref[0, 0, :] = idx


def _tc_dist_argmin(xf, embt):
    return pl.pallas_call(
        _dist_argmin_body,
        grid=(NT,),
        in_specs=[
            pl.BlockSpec((BN, DP), lambda i: (i, 0)),
            pl.BlockSpec((DP, K), lambda i: (0, 0)),
        ],
        out_specs=pl.BlockSpec((1, 1, BN), lambda i: (i, 0, 0)),
        out_shape=jax.ShapeDtypeStruct((NT, 1, BN), jnp.int32),
        compiler_params=pltpu.CompilerParams(
            dimension_semantics=("arbitrary",),
        ),
    )(xf, embt)


@functools.cache
def _make_sc_gather_loss():
    return pl.kernel(
        _sc_gather_loss_body,
        mesh=plsc.VectorSubcoreMesh(core_axis_name="c", subcore_axis_name="s"),
        out_type=[
            jax.ShapeDtypeStruct((N * D,), jnp.float32),
            jax.ShapeDtypeStruct((NW * 16,), jnp.float32),
        ],
        scratch_types=[
            pltpu.VMEM((RPW,), jnp.int32),
            pltpu.VMEM((RPW * D,), jnp.int32),
            pltpu.VMEM((RPW * D,), jnp.float32),
            pltpu.VMEM((RPW * D,), jnp.float32),
            pltpu.VMEM((16,), jnp.float32),
            pltpu.SemaphoreType.DMA,
        ],
    )


def _sc_gather_loss_body(emb_hbm, idx_hbm, xf_hbm, out_hbm, part_hbm,
                         idx_v, widx_v, rows_v, xf_v, acc_v, sem):
    wid = lax.axis_index("s") * 2 + lax.axis_index("c")
    base = wid * RPW
    pltpu.sync_copy(idx_hbm.at[pl.ds(base, RPW)], idx_v)
    pltpu.sync_copy(xf_hbm.at[pl.ds(base * D, RPW * D)], xf_v)

    lane = lax.iota(jnp.int32, 16)
    rep = lane >> 3
    off = lane & 7
    dnums = lax.GatherDimensionNumbers(
        offset_dims=(), collapsed_slice_dims=(0,), start_index_map=(0,))

    def tile_body(t, _):
        tile = idx_v[pl.ds(t * 16, 16)]
        for c in range(8):
            vals = lax.gather(
                tile, (2 * c + rep)[:, None], dnums, (1,),
                mode=lax.GatherScatterMode.PROMISE_IN_BOUNDS)
            widx_v[pl.ds(t * 128 + c * 16, 16)] = vals * 8 + off
        return 0

    lax.fori_loop(0, RPW // 16, tile_body, 0)

    copies = []
    for g in range(RPW * D // IDX_CHUNK):
        copies.append(pltpu.async_copy(
            emb_hbm.at[widx_v.at[pl.ds(g * IDX_CHUNK, IDX_CHUNK)]],
            rows_v.at[pl.ds(g * IDX_CHUNK, IDX_CHUNK)],
            sem,
        ))
    for c in copies:
        c.wait()

    def body(i, acc):
        dlt = rows_v[pl.ds(i * 16, 16)] - xf_v[pl.ds(i * 16, 16)]
        return acc + dlt * dlt

    acc = lax.fori_loop(0, RPW * D // 16, body, jnp.zeros((16,), jnp.float32))
    acc_v[...] = acc
    pltpu.sync_copy(rows_v, out_hbm.at[pl.ds(base * D, RPW * D)])
    pltpu.sync_copy(acc_v, part_hbm.at[pl.ds(wid * 16, 16)])


def kernel(x, embeddings):
    xf = x.reshape(N, D)
    xf2_p = jnp.pad(xf + xf, ((0, 0), (0, DP - D)))
    embt_p = jnp.pad(embeddings.T, ((0, DP - D), (0, 0)))
    idx3 = _tc_dist_argmin(xf2_p, embt_p)
    indices = idx3.reshape(N)
    q, parts = _make_sc_gather_loss()(embeddings.reshape(-1), indices,
                                      xf.reshape(-1))
    m = jnp.sum(parts) / jnp.float32(N * D)
    loss = m + jnp.float32(0.25) * m
    quantized = q.reshape(x.shape)
    return (quantized, loss, indices)
